# initial kernel scaffold (unmeasured)
import functools

import jax
import jax.numpy as jnp
from jax import lax
from jax.experimental import pallas as pl
from jax.experimental.pallas import tpu as pltpu

N_DEV = 16
B = 2
SQ = 128
HQ = 4
DH = 64
D_MODEL = 512
SKV = N_DEV * SQ
NEG = -1e9


def kernel(x, Wq, K_ext, V_ext, Wo):
    def body(x_ref, wq_ref, k_ref, v_ref, wo_ref, out_ref,
             kv_buf, send_sems, recv_sems):
        my = lax.axis_index("i")
        left = lax.rem(my + N_DEV - 1, N_DEV)
        right = lax.rem(my + 1, N_DEV)

        barrier_sem = pltpu.get_barrier_semaphore()
        for nbr in (left, right):
            pl.semaphore_signal(barrier_sem, inc=1, device_id=(nbr,),
                                device_id_type=pl.DeviceIdType.MESH)
        pl.semaphore_wait(barrier_sem, 2)

        kv_buf[my, 0] = k_ref[...].astype(jnp.bfloat16)
        kv_buf[my, 1] = v_ref[...].astype(jnp.bfloat16)

        for h in range(N_DEV - 1):
            send_ok = jnp.logical_and(my - h >= 0, my < N_DEV - 1)
            recv_ok = my - h >= 1

            @pl.when(send_ok)
            def _(h=h):
                rdma = pltpu.make_async_remote_copy(
                    src_ref=kv_buf.at[my - h],
                    dst_ref=kv_buf.at[my - h],
                    send_sem=send_sems.at[h],
                    recv_sem=recv_sems.at[h],
                    device_id=(my + 1,),
                    device_id_type=pl.DeviceIdType.MESH,
                )
                rdma.start()
                rdma.wait_send()

            @pl.when(recv_ok)
            def _(h=h):
                rdma = pltpu.make_async_remote_copy(
                    src_ref=kv_buf.at[my - 1 - h],
                    dst_ref=kv_buf.at[my - 1 - h],
                    send_sem=send_sems.at[h],
                    recv_sem=recv_sems.at[h],
                    device_id=(my + 1,),
                    device_id_type=pl.DeviceIdType.MESH,
                )
                rdma.wait_recv()

        rows = lax.broadcasted_iota(jnp.int32, (SQ, SKV), 0)
        cols = lax.broadcasted_iota(jnp.int32, (SQ, SKV), 1)
        qb = 2 * my + rows // 64
        kb = cols // 64
        mask = kb <= qb

        wq = wq_ref[...].astype(jnp.bfloat16)
        for b in range(B):
            xb = x_ref[b].astype(jnp.bfloat16)
            q = lax.dot_general(
                xb, wq, (((1,), (0,)), ((), ())),
                preferred_element_type=jnp.float32,
            )
            acc = jnp.zeros((SQ, D_MODEL), jnp.float32)
            for hh in range(HQ):
                qh = q[:, hh * DH:(hh + 1) * DH].astype(jnp.bfloat16)
                kh = kv_buf[:, 0, b, :, hh, :].reshape(SKV, DH)
                s = lax.dot_general(
                    qh, kh, (((1,), (1,)), ((), ())),
                    preferred_element_type=jnp.float32,
                ) * 0.125
                s = jnp.where(mask, s, NEG)
                m = jnp.max(s, axis=1, keepdims=True)
                w = jnp.exp(s - m)
                w = w / jnp.sum(w, axis=1, keepdims=True)
                vh = kv_buf[:, 1, b, :, hh, :].reshape(SKV, DH)
                ctx = lax.dot_general(
                    w.astype(jnp.bfloat16), vh, (((1,), (0,)), ((), ())),
                    preferred_element_type=jnp.float32,
                )
                wo_h = wo_ref[hh * DH:(hh + 1) * DH, :].astype(jnp.bfloat16)
                acc = acc + lax.dot_general(
                    ctx.astype(jnp.bfloat16), wo_h, (((1,), (0,)), ((), ())),
                    preferred_element_type=jnp.float32,
                )
            out_ref[b] = acc

        @functools.partial(pl.run_scoped,
                           second_barrier=pltpu.SemaphoreType.REGULAR)
        def _(second_barrier):
            for nbr in (left, right):
                pl.semaphore_signal(second_barrier, inc=1, device_id=(nbr,),
                                    device_id_type=pl.DeviceIdType.MESH)
            pl.semaphore_wait(second_barrier, 2)

    return pl.pallas_call(
        body,
        out_shape=jax.ShapeDtypeStruct((B, SQ, D_MODEL), jnp.float32),
        in_specs=[pl.BlockSpec(memory_space=pltpu.VMEM)] * 5,
        out_specs=pl.BlockSpec(memory_space=pltpu.VMEM),
        scratch_shapes=[
            pltpu.VMEM((N_DEV, 2, B, SQ, HQ, DH), jnp.bfloat16),
            pltpu.SemaphoreType.DMA((N_DEV - 1,)),
            pltpu.SemaphoreType.DMA((N_DEV - 1,)),
        ],
        compiler_params=pltpu.CompilerParams(collective_id=0),
    )(x, Wq, K_ext, V_ext, Wo)


# baseline (device time: 148081 ns/iter reference)
import functools

import jax
import jax.numpy as jnp
from jax import lax
from jax.experimental import pallas as pl
from jax.experimental.pallas import tpu as pltpu

N_DEV = 16
B = 2
SQ = 128
HQ = 4
DH = 64
D_MODEL = 512
SKV = N_DEV * SQ
NEG = -1e9


def kernel(x, Wq, K_ext, V_ext, Wo):
    def body(x_ref, wq_ref, k_ref, v_ref, wo_ref, out_ref,
             kv_buf, send_sems, recv_sems):
        my = lax.axis_index("i")
        left = lax.rem(my + N_DEV - 1, N_DEV)
        right = lax.rem(my + 1, N_DEV)

        kv_buf[...] = jnp.zeros(
            (N_DEV, 2, B, SQ, HQ, DH), jnp.bfloat16)

        barrier_sem = pltpu.get_barrier_semaphore()
        for nbr in (left, right):
            pl.semaphore_signal(barrier_sem, inc=1, device_id=(nbr,),
                                device_id_type=pl.DeviceIdType.MESH)
        pl.semaphore_wait(barrier_sem, 2)

        kv_buf[my, 0] = k_ref[...].astype(jnp.bfloat16)
        kv_buf[my, 1] = v_ref[...].astype(jnp.bfloat16)

        for h in range(N_DEV - 1):
            send_ok = jnp.logical_and(my - h >= 0, my < N_DEV - 1)
            recv_ok = my - h >= 1

            @pl.when(send_ok)
            def _(h=h):
                rdma = pltpu.make_async_remote_copy(
                    src_ref=kv_buf.at[my - h],
                    dst_ref=kv_buf.at[my - h],
                    send_sem=send_sems.at[h],
                    recv_sem=recv_sems.at[h],
                    device_id=(my + 1,),
                    device_id_type=pl.DeviceIdType.MESH,
                )
                rdma.start()
                rdma.wait_send()

            @pl.when(recv_ok)
            def _(h=h):
                rdma = pltpu.make_async_remote_copy(
                    src_ref=kv_buf.at[my - 1 - h],
                    dst_ref=kv_buf.at[my - 1 - h],
                    send_sem=send_sems.at[h],
                    recv_sem=recv_sems.at[h],
                    device_id=(my + 1,),
                    device_id_type=pl.DeviceIdType.MESH,
                )
                rdma.wait_recv()

        rows = lax.broadcasted_iota(jnp.int32, (SQ, SKV), 0)
        cols = lax.broadcasted_iota(jnp.int32, (SQ, SKV), 1)
        qb = 2 * my + rows // 64
        kb = cols // 64
        mask = kb <= qb

        wq = wq_ref[...].astype(jnp.bfloat16)
        for b in range(B):
            xb = x_ref[b].astype(jnp.bfloat16)
            q = lax.dot_general(
                xb, wq, (((1,), (0,)), ((), ())),
                preferred_element_type=jnp.float32,
            )
            acc = jnp.zeros((SQ, D_MODEL), jnp.float32)
            for hh in range(HQ):
                qh = q[:, hh * DH:(hh + 1) * DH].astype(jnp.bfloat16)
                kh = kv_buf[:, 0, b, :, hh, :].reshape(SKV, DH)
                s = lax.dot_general(
                    qh, kh, (((1,), (1,)), ((), ())),
                    preferred_element_type=jnp.float32,
                ) * 0.125
                s = jnp.where(mask, s, NEG)
                m = jnp.max(s, axis=1, keepdims=True)
                w = jnp.exp(s - m)
                w = w / jnp.sum(w, axis=1, keepdims=True)
                vh = kv_buf[:, 1, b, :, hh, :].reshape(SKV, DH)
                ctx = lax.dot_general(
                    w.astype(jnp.bfloat16), vh, (((1,), (0,)), ((), ())),
                    preferred_element_type=jnp.float32,
                )
                wo_h = wo_ref[hh * DH:(hh + 1) * DH, :].astype(jnp.bfloat16)
                acc = acc + lax.dot_general(
                    ctx.astype(jnp.bfloat16), wo_h, (((1,), (0,)), ((), ())),
                    preferred_element_type=jnp.float32,
                )
            out_ref[b] = acc

        @functools.partial(pl.run_scoped,
                           second_barrier=pltpu.SemaphoreType.REGULAR)
        def _(second_barrier):
            for nbr in (left, right):
                pl.semaphore_signal(second_barrier, inc=1, device_id=(nbr,),
                                    device_id_type=pl.DeviceIdType.MESH)
            pl.semaphore_wait(second_barrier, 2)

    return pl.pallas_call(
        body,
        out_shape=jax.ShapeDtypeStruct((B, SQ, D_MODEL), jnp.float32),
        in_specs=[pl.BlockSpec(memory_space=pltpu.VMEM)] * 5,
        out_specs=pl.BlockSpec(memory_space=pltpu.VMEM),
        scratch_shapes=[
            pltpu.VMEM((N_DEV, 2, B, SQ, HQ, DH), jnp.bfloat16),
            pltpu.SemaphoreType.DMA((N_DEV - 1,)),
            pltpu.SemaphoreType.DMA((N_DEV - 1,)),
        ],
        compiler_params=pltpu.CompilerParams(collective_id=0),
    )(x, Wq, K_ext, V_ext, Wo)
